# Initial kernel scaffold; baseline (speedup 1.0000x reference)
import jax
import jax.numpy as jnp
from jax.experimental import pallas as pl


def kernel(x, edge_index, W1, b1, W2, b2):
    raise NotImplementedError("write your pallas kernel here")


# speedup vs baseline: 23.9609x; 23.9609x over previous
import functools

import jax
import jax.numpy as jnp
from jax import lax
from jax.experimental import pallas as pl
from jax.experimental.pallas import tpu as pltpu
from jax.experimental.pallas import tpu_sc as plsc

N = 10000
NPAD = 10240
E = 320000
NC, NS = 2, 16
CH = 128
K = 79
EPAD = NC * NS * K * CH
ROWS_PER_TILE = NPAD // NS
ZR = 64


def _sc_mesh():
    return plsc.VectorSubcoreMesh(
        core_axis_name="c", subcore_axis_name="s", num_cores=NC, num_subcores=NS)


def _make_deg_kernel():
    @functools.partial(
        pl.kernel,
        out_type=jax.ShapeDtypeStruct((NC, NPAD, 8), jnp.float32),
        mesh=_sc_mesh(),
        compiler_params=pltpu.CompilerParams(use_tc_tiling_on_sc=False),
        scratch_types=[
            pltpu.VMEM((K, CH), jnp.int32),
            pltpu.VMEM((CH, 8), jnp.float32),
            pltpu.VMEM((ZR, 8), jnp.float32),
            pltpu.VMEM_SHARED((NPAD, 8), jnp.float32),
        ],
    )
    def deg_kernel(dst_hbm, ones_hbm, zz_hbm, out_hbm, dst_v, ones_v, stage_v, acc_sh):
        c = lax.axis_index("c")
        s = lax.axis_index("s")
        wid = c * NS + s
        pltpu.sync_copy(zz_hbm, stage_v)
        def zbody(i, carry):
            pltpu.sync_copy(stage_v, acc_sh.at[pl.ds(s * ROWS_PER_TILE + i * ZR, ZR)])
            return carry
        lax.fori_loop(0, ROWS_PER_TILE // ZR, zbody, 0)
        pltpu.sync_copy(ones_hbm, ones_v)
        pltpu.sync_copy(dst_hbm.at[wid], dst_v)
        plsc.subcore_barrier()
        def ebody(j, carry):
            pltpu.sync_copy(ones_v, acc_sh.at[dst_v.at[j]], add=True)
            return carry
        lax.fori_loop(0, K, ebody, 0)
        plsc.subcore_barrier()
        def obody(i, carry):
            r0 = s * ROWS_PER_TILE + i * ZR
            pltpu.sync_copy(acc_sh.at[pl.ds(r0, ZR)], stage_v)
            pltpu.sync_copy(stage_v, out_hbm.at[c, pl.ds(r0, ZR)])
            return carry
        lax.fori_loop(0, ROWS_PER_TILE // ZR, obody, 0)

    return deg_kernel


def _make_agg_kernel(d):
    @functools.partial(
        pl.kernel,
        out_type=jax.ShapeDtypeStruct((NC, NPAD, d), jnp.float32),
        mesh=_sc_mesh(),
        compiler_params=pltpu.CompilerParams(use_tc_tiling_on_sc=False),
        scratch_types=[
            pltpu.VMEM((K, CH), jnp.int32),
            pltpu.VMEM((K, CH), jnp.int32),
            pltpu.VMEM((CH, d), jnp.float32),
            pltpu.VMEM((ZR, d), jnp.float32),
            pltpu.VMEM_SHARED((NPAD, d), jnp.float32),
            pltpu.SemaphoreType.DMA,
        ],
    )
    def agg_kernel(src_hbm, dst_hbm, h_hbm, zz_hbm, out_hbm,
                   src_v, dst_v, rows_v, stage_v, acc_sh, sem):
        c = lax.axis_index("c")
        s = lax.axis_index("s")
        wid = c * NS + s
        pltpu.sync_copy(zz_hbm, stage_v)
        def zbody(i, carry):
            pltpu.sync_copy(stage_v, acc_sh.at[pl.ds(s * ROWS_PER_TILE + i * ZR, ZR)])
            return carry
        lax.fori_loop(0, ROWS_PER_TILE // ZR, zbody, 0)
        pltpu.sync_copy(src_hbm.at[wid], src_v)
        pltpu.sync_copy(dst_hbm.at[wid], dst_v)
        plsc.subcore_barrier()
        def ebody(j, carry):
            pltpu.async_copy(h_hbm.at[src_v.at[j]], rows_v, sem).wait()
            pltpu.sync_copy(rows_v, acc_sh.at[dst_v.at[j]], add=True)
            return carry
        lax.fori_loop(0, K, ebody, 0)
        plsc.subcore_barrier()
        def obody(i, carry):
            r0 = s * ROWS_PER_TILE + i * ZR
            pltpu.sync_copy(acc_sh.at[pl.ds(r0, ZR)], stage_v)
            pltpu.sync_copy(stage_v, out_hbm.at[c, pl.ds(r0, ZR)])
            return carry
        lax.fori_loop(0, ROWS_PER_TILE // ZR, obody, 0)

    return agg_kernel


_BLK = 1000
_GRID = N // _BLK


def _dis_from(dg):
    deg = dg[0, :, 0] + dg[1, :, 0] + 1.0
    return lax.rsqrt(deg)


def _mm_scale_body(x_ref, w_ref, dg_ref, o_ref):
    dis = _dis_from(dg_ref[...])
    h = jnp.dot(x_ref[...], w_ref[...], preferred_element_type=jnp.float32)
    o_ref[...] = h * dis[:, None]


def _mid_body(agg_ref, hp_ref, dg_ref, w2_ref, b1_ref, o_ref):
    dis = _dis_from(dg_ref[...])
    agg = agg_ref[...]
    h1 = dis[:, None] * (agg[0] + agg[1] + hp_ref[...]) + b1_ref[...]
    h1 = jnp.maximum(h1, 0.0)
    o_ref[...] = jnp.dot(h1, w2_ref[...], preferred_element_type=jnp.float32) * dis[:, None]


def _fin_body(agg_ref, hp_ref, dg_ref, b2_ref, o_ref):
    dis = _dis_from(dg_ref[...])
    agg = agg_ref[...]
    o_ref[...] = dis[:, None] * (agg[0] + agg[1] + hp_ref[...]) + b2_ref[...]


def _row_spec(d):
    return pl.BlockSpec((_BLK, d), lambda i: (i, 0))


def _dg_spec():
    return pl.BlockSpec((2, _BLK, 8), lambda i: (0, i, 0))


def _full_spec(r, cdim):
    return pl.BlockSpec((r, cdim), lambda i: (0, 0))


def _agg_spec(d):
    return pl.BlockSpec((2, _BLK, d), lambda i: (0, i, 0))


@jax.jit
def kernel(x, edge_index, W1, b1, W2, b2):
    d_hid = W1.shape[1]
    d_out = W2.shape[1]

    npad = EPAD - E
    pad = jnp.arange(npad, dtype=jnp.int32)
    src_p = jnp.concatenate([edge_index[0], pad % N]).reshape(NC * NS, K, CH)
    dst_p = jnp.concatenate([edge_index[1], N + pad % (NPAD - N)]).reshape(NC * NS, K, CH)

    ones8 = jnp.ones((CH, 8), jnp.float32)
    z8 = jnp.zeros((ZR, 8), jnp.float32)
    zh = jnp.zeros((ZR, d_hid), jnp.float32)
    zo = jnp.zeros((ZR, d_out), jnp.float32)

    degw = _make_deg_kernel()(dst_p, ones8, z8)
    dg = degw[:, :N, :]

    h1p = pl.pallas_call(
        _mm_scale_body,
        grid=(_GRID,),
        in_specs=[_row_spec(x.shape[1]), _full_spec(x.shape[1], d_hid), _dg_spec()],
        out_specs=_row_spec(d_hid),
        out_shape=jax.ShapeDtypeStruct((N, d_hid), jnp.float32),
    )(x, W1, dg)

    agg1 = _make_agg_kernel(d_hid)(src_p, dst_p, h1p, zh)[:, :N, :]

    h2p = pl.pallas_call(
        _mid_body,
        grid=(_GRID,),
        in_specs=[_agg_spec(d_hid), _row_spec(d_hid), _dg_spec(),
                  _full_spec(d_hid, d_out), _full_spec(1, d_hid)],
        out_specs=_row_spec(d_out),
        out_shape=jax.ShapeDtypeStruct((N, d_out), jnp.float32),
    )(agg1, h1p, dg, W2, b1.reshape(1, d_hid))

    agg2 = _make_agg_kernel(d_out)(src_p, dst_p, h2p, zo)[:, :N, :]

    out = pl.pallas_call(
        _fin_body,
        grid=(_GRID,),
        in_specs=[_agg_spec(d_out), _row_spec(d_out), _dg_spec(),
                  _full_spec(1, d_out)],
        out_specs=_row_spec(d_out),
        out_shape=jax.ShapeDtypeStruct((N, d_out), jnp.float32),
    )(agg2, h2p, dg, b2.reshape(1, d_out))

    return out


# speedup vs baseline: 32.1955x; 1.3437x over previous
import functools

import jax
import jax.numpy as jnp
from jax import lax
from jax.experimental import pallas as pl
from jax.experimental.pallas import tpu as pltpu
from jax.experimental.pallas import tpu_sc as plsc

N = 10000
NPAD = 10240
E = 320000
NC, NS = 2, 16
CH = 128
K = 80
PH = 2
K2 = K // PH
EPAD = NC * NS * K * CH
ROWS_PER_TILE = NPAD // NS
ZR = 64


def _sc_mesh():
    return plsc.VectorSubcoreMesh(
        core_axis_name="c", subcore_axis_name="s", num_cores=NC, num_subcores=NS)


def _make_deg_kernel():
    @functools.partial(
        pl.kernel,
        out_type=jax.ShapeDtypeStruct((NC, NPAD, 8), jnp.float32),
        mesh=_sc_mesh(),
        compiler_params=pltpu.CompilerParams(use_tc_tiling_on_sc=False),
        scratch_types=[
            pltpu.VMEM((K, CH), jnp.int32),
            pltpu.VMEM((CH, 8), jnp.float32),
            pltpu.VMEM((ZR, 8), jnp.float32),
            pltpu.VMEM_SHARED((NPAD, 8), jnp.float32),
        ],
    )
    def deg_kernel(dst_hbm, ones_hbm, zz_hbm, out_hbm, dst_v, ones_v, stage_v, acc_sh):
        c = lax.axis_index("c")
        s = lax.axis_index("s")
        wid = c * NS + s
        pltpu.sync_copy(zz_hbm, stage_v)
        def zbody(i, carry):
            pltpu.sync_copy(stage_v, acc_sh.at[pl.ds(s * ROWS_PER_TILE + i * ZR, ZR)])
            return carry
        lax.fori_loop(0, ROWS_PER_TILE // ZR, zbody, 0)
        pltpu.sync_copy(ones_hbm, ones_v)
        pltpu.sync_copy(dst_hbm.at[wid], dst_v)
        plsc.subcore_barrier()
        def ebody(j, carry):
            pltpu.sync_copy(ones_v, acc_sh.at[dst_v.at[j]], add=True)
            return carry
        lax.fori_loop(0, K, ebody, 0)
        plsc.subcore_barrier()
        def obody(i, carry):
            r0 = s * ROWS_PER_TILE + i * ZR
            pltpu.sync_copy(acc_sh.at[pl.ds(r0, ZR)], stage_v)
            pltpu.sync_copy(stage_v, out_hbm.at[c, pl.ds(r0, ZR)])
            return carry
        lax.fori_loop(0, ROWS_PER_TILE // ZR, obody, 0)

    return deg_kernel


def _make_agg_kernel(d):
    @functools.partial(
        pl.kernel,
        out_type=jax.ShapeDtypeStruct((NC, NPAD, d), jnp.float32),
        mesh=_sc_mesh(),
        compiler_params=pltpu.CompilerParams(use_tc_tiling_on_sc=False),
        scratch_types=[
            pltpu.VMEM((K2, CH), jnp.int32),
            pltpu.VMEM((K2, CH), jnp.int32),
            pltpu.VMEM((CH, d), jnp.float32),
            pltpu.VMEM((CH, d), jnp.float32),
            pltpu.VMEM_SHARED((NPAD, d), jnp.float32),
            pltpu.SemaphoreType.DMA,
            pltpu.SemaphoreType.DMA,
        ],
    )
    def agg_kernel(src_hbm, dst_hbm, h_hbm, zz_hbm, out_hbm,
                   src_v, dst_v, rows_a, rows_b, acc_sh, sem_a, sem_b):
        c = lax.axis_index("c")
        s = lax.axis_index("s")
        wid = c * NS + s
        pltpu.sync_copy(zz_hbm, rows_a)
        for i in range(ROWS_PER_TILE // CH):
            pltpu.sync_copy(rows_a, acc_sh.at[pl.ds(s * ROWS_PER_TILE + i * CH, CH)])
        plsc.subcore_barrier()
        for p in range(PH):
            pltpu.sync_copy(src_hbm.at[wid, pl.ds(p * K2, K2)], src_v)
            pltpu.sync_copy(dst_hbm.at[wid, pl.ds(p * K2, K2)], dst_v)
            pltpu.async_copy(h_hbm.at[src_v.at[0]], rows_a, sem_a)
            def ebody(i, carry):
                c0 = 2 * i
                c1 = 2 * i + 1
                pltpu.async_copy(h_hbm.at[src_v.at[c1]], rows_b, sem_b)
                pltpu.make_async_copy(h_hbm.at[src_v.at[c0]], rows_a, sem_a).wait()
                pltpu.sync_copy(rows_a, acc_sh.at[dst_v.at[c0]], add=True)
                nxt = jnp.minimum(c0 + 2, K2 - 1)
                pltpu.async_copy(h_hbm.at[src_v.at[nxt]], rows_a, sem_a)
                pltpu.make_async_copy(h_hbm.at[src_v.at[c1]], rows_b, sem_b).wait()
                pltpu.sync_copy(rows_b, acc_sh.at[dst_v.at[c1]], add=True)
                return carry
            lax.fori_loop(0, K2 // 2, ebody, 0)
            pltpu.make_async_copy(h_hbm.at[src_v.at[K2 - 1]], rows_a, sem_a).wait()
        plsc.subcore_barrier()
        for i in range(ROWS_PER_TILE // CH):
            r0 = s * ROWS_PER_TILE + i * CH
            pltpu.sync_copy(acc_sh.at[pl.ds(r0, CH)], rows_a)
            pltpu.sync_copy(rows_a, out_hbm.at[c, pl.ds(r0, CH)])

    return agg_kernel


_BLK = 1000
_GRID = N // _BLK


def _dis_from(dg):
    deg = dg[0, :, 0] + dg[1, :, 0] + 1.0
    return lax.rsqrt(deg)


def _mm_scale_body(x_ref, w_ref, dg_ref, o_ref):
    dis = _dis_from(dg_ref[...])
    h = jnp.dot(x_ref[...], w_ref[...], preferred_element_type=jnp.float32)
    o_ref[...] = h * dis[:, None]


def _mid_body(agg_ref, hp_ref, dg_ref, w2_ref, b1_ref, o_ref):
    dis = _dis_from(dg_ref[...])
    agg = agg_ref[...]
    h1 = dis[:, None] * (agg[0] + agg[1] + hp_ref[...]) + b1_ref[...]
    h1 = jnp.maximum(h1, 0.0)
    o_ref[...] = jnp.dot(h1, w2_ref[...], preferred_element_type=jnp.float32) * dis[:, None]


def _fin_body(agg_ref, hp_ref, dg_ref, b2_ref, o_ref):
    dis = _dis_from(dg_ref[...])
    agg = agg_ref[...]
    o_ref[...] = dis[:, None] * (agg[0] + agg[1] + hp_ref[...]) + b2_ref[...]


def _row_spec(d):
    return pl.BlockSpec((_BLK, d), lambda i: (i, 0))


def _dg_spec():
    return pl.BlockSpec((2, _BLK, 8), lambda i: (0, i, 0))


def _full_spec(r, cdim):
    return pl.BlockSpec((r, cdim), lambda i: (0, 0))


def _agg_spec(d):
    return pl.BlockSpec((2, _BLK, d), lambda i: (0, i, 0))


@jax.jit
def kernel(x, edge_index, W1, b1, W2, b2):
    d_hid = W1.shape[1]
    d_out = W2.shape[1]

    npad = EPAD - E
    pad = jnp.arange(npad, dtype=jnp.int32)
    src_p = jnp.concatenate([edge_index[0], pad % N]).reshape(NC * NS, K, CH)
    dst_p = jnp.concatenate([edge_index[1], N + pad % (NPAD - N)]).reshape(NC * NS, K, CH)

    ones8 = jnp.ones((CH, 8), jnp.float32)
    z8 = jnp.zeros((ZR, 8), jnp.float32)
    zh = jnp.zeros((CH, d_hid), jnp.float32)
    zo = jnp.zeros((CH, d_out), jnp.float32)

    degw = _make_deg_kernel()(dst_p, ones8, z8)
    dg = degw[:, :N, :]

    h1p = pl.pallas_call(
        _mm_scale_body,
        grid=(_GRID,),
        in_specs=[_row_spec(x.shape[1]), _full_spec(x.shape[1], d_hid), _dg_spec()],
        out_specs=_row_spec(d_hid),
        out_shape=jax.ShapeDtypeStruct((N, d_hid), jnp.float32),
    )(x, W1, dg)

    agg1 = _make_agg_kernel(d_hid)(src_p, dst_p, h1p, zh)[:, :N, :]

    h2p = pl.pallas_call(
        _mid_body,
        grid=(_GRID,),
        in_specs=[_agg_spec(d_hid), _row_spec(d_hid), _dg_spec(),
                  _full_spec(d_hid, d_out), _full_spec(1, d_hid)],
        out_specs=_row_spec(d_out),
        out_shape=jax.ShapeDtypeStruct((N, d_out), jnp.float32),
    )(agg1, h1p, dg, W2, b1.reshape(1, d_hid))

    agg2 = _make_agg_kernel(d_out)(src_p, dst_p, h2p, zo)[:, :N, :]

    out = pl.pallas_call(
        _fin_body,
        grid=(_GRID,),
        in_specs=[_agg_spec(d_out), _row_spec(d_out), _dg_spec(),
                  _full_spec(1, d_out)],
        out_specs=_row_spec(d_out),
        out_shape=jax.ShapeDtypeStruct((N, d_out), jnp.float32),
    )(agg2, h2p, dg, b2.reshape(1, d_out))

    return out


# speedup vs baseline: 34.3134x; 1.0658x over previous
import functools

import jax
import jax.numpy as jnp
from jax import lax
from jax.experimental import pallas as pl
from jax.experimental.pallas import tpu as pltpu
from jax.experimental.pallas import tpu_sc as plsc

N = 10000
NPAD = 10240
E = 320000
NC, NS = 2, 16
CH = 128
K = 80
PH = 2
K2 = K // PH
EPAD = NC * NS * K * CH
ROWS_PER_TILE = NPAD // NS
ZR = 64


def _sc_mesh():
    return plsc.VectorSubcoreMesh(
        core_axis_name="c", subcore_axis_name="s", num_cores=NC, num_subcores=NS)


def _make_deg_kernel():
    @functools.partial(
        pl.kernel,
        out_type=jax.ShapeDtypeStruct((NC, NPAD, 8), jnp.float32),
        mesh=_sc_mesh(),
        compiler_params=pltpu.CompilerParams(use_tc_tiling_on_sc=False),
        scratch_types=[
            pltpu.VMEM((K, CH), jnp.int32),
            pltpu.VMEM((CH, 8), jnp.float32),
            pltpu.VMEM((ZR, 8), jnp.float32),
            pltpu.VMEM_SHARED((NPAD, 8), jnp.float32),
        ],
    )
    def deg_kernel(dst_hbm, ones_hbm, zz_hbm, out_hbm, dst_v, ones_v, stage_v, acc_sh):
        c = lax.axis_index("c")
        s = lax.axis_index("s")
        wid = c * NS + s
        pltpu.sync_copy(zz_hbm, stage_v)
        def zbody(i, carry):
            pltpu.sync_copy(stage_v, acc_sh.at[pl.ds(s * ROWS_PER_TILE + i * ZR, ZR)])
            return carry
        lax.fori_loop(0, ROWS_PER_TILE // ZR, zbody, 0)
        pltpu.sync_copy(ones_hbm, ones_v)
        pltpu.sync_copy(dst_hbm.at[wid], dst_v)
        plsc.subcore_barrier()
        def ebody(j, carry):
            pltpu.sync_copy(ones_v, acc_sh.at[dst_v.at[j]], add=True)
            return carry
        lax.fori_loop(0, K, ebody, 0)
        plsc.subcore_barrier()
        def obody(i, carry):
            r0 = s * ROWS_PER_TILE + i * ZR
            pltpu.sync_copy(acc_sh.at[pl.ds(r0, ZR)], stage_v)
            pltpu.sync_copy(stage_v, out_hbm.at[c, pl.ds(r0, ZR)])
            return carry
        lax.fori_loop(0, ROWS_PER_TILE // ZR, obody, 0)

    return deg_kernel


def _make_agg_kernel(d):
    @functools.partial(
        pl.kernel,
        out_type=jax.ShapeDtypeStruct((NC, NPAD, d), jnp.float32),
        mesh=_sc_mesh(),
        compiler_params=pltpu.CompilerParams(use_tc_tiling_on_sc=False),
        scratch_types=[
            pltpu.VMEM((K2, CH), jnp.int32),
            pltpu.VMEM((K2, CH), jnp.int32),
            pltpu.VMEM((CH, d), jnp.float32),
            pltpu.VMEM((CH, d), jnp.float32),
            pltpu.VMEM_SHARED((NPAD, d), jnp.float32),
            pltpu.SemaphoreType.DMA,
            pltpu.SemaphoreType.DMA,
        ],
    )
    def agg_kernel(src_hbm, dst_hbm, h_hbm, zz_hbm, out_hbm,
                   src_v, dst_v, rows_a, rows_b, acc_sh, sem_a, sem_b):
        c = lax.axis_index("c")
        s = lax.axis_index("s")
        wid = c * NS + s
        pltpu.sync_copy(zz_hbm, rows_a)
        for i in range(ROWS_PER_TILE // CH):
            pltpu.sync_copy(rows_a, acc_sh.at[pl.ds(s * ROWS_PER_TILE + i * CH, CH)])
        plsc.subcore_barrier()
        for p in range(PH):
            pltpu.sync_copy(src_hbm.at[wid, pl.ds(p * K2, K2)], src_v)
            pltpu.sync_copy(dst_hbm.at[wid, pl.ds(p * K2, K2)], dst_v)
            pltpu.async_copy(h_hbm.at[src_v.at[0]], rows_a, sem_a)
            def ebody(i, carry):
                c0 = 2 * i
                c1 = 2 * i + 1
                pltpu.async_copy(h_hbm.at[src_v.at[c1]], rows_b, sem_b)
                pltpu.make_async_copy(h_hbm.at[src_v.at[c0]], rows_a, sem_a).wait()
                pltpu.sync_copy(rows_a, acc_sh.at[dst_v.at[c0]], add=True)
                nxt = jnp.minimum(c0 + 2, K2 - 1)
                pltpu.async_copy(h_hbm.at[src_v.at[nxt]], rows_a, sem_a)
                pltpu.make_async_copy(h_hbm.at[src_v.at[c1]], rows_b, sem_b).wait()
                pltpu.sync_copy(rows_b, acc_sh.at[dst_v.at[c1]], add=True)
                return carry
            lax.fori_loop(0, K2 // 2, ebody, 0)
            pltpu.make_async_copy(h_hbm.at[src_v.at[K2 - 1]], rows_a, sem_a).wait()
        plsc.subcore_barrier()
        for i in range(ROWS_PER_TILE // CH):
            r0 = s * ROWS_PER_TILE + i * CH
            pltpu.sync_copy(acc_sh.at[pl.ds(r0, CH)], rows_a)
            pltpu.sync_copy(rows_a, out_hbm.at[c, pl.ds(r0, CH)])

    return agg_kernel


_BLK = 1000
_GRID = N // _BLK


def _dis_from(dg):
    deg = dg[0, :, 0] + dg[1, :, 0] + 1.0
    return lax.rsqrt(deg)


def _mm_scale_body(x_ref, w_ref, dg_ref, o_ref):
    dis = _dis_from(dg_ref[...])
    h = jnp.dot(x_ref[...], w_ref[...], preferred_element_type=jnp.float32)
    o_ref[...] = h * dis[:, None]


def _mid_body(agg_ref, hp_ref, dg_ref, w2_ref, b1_ref, o_ref):
    dis = _dis_from(dg_ref[...])
    agg = agg_ref[...]
    h1 = dis[:, None] * (agg[0] + agg[1] + hp_ref[...]) + b1_ref[...]
    h1 = jnp.maximum(h1, 0.0)
    o_ref[...] = jnp.dot(h1, w2_ref[...], preferred_element_type=jnp.float32) * dis[:, None]


def _fin_body(agg_ref, hp_ref, dg_ref, b2_ref, o_ref):
    dis = _dis_from(dg_ref[...])
    agg = agg_ref[...]
    o_ref[...] = dis[:, None] * (agg[0] + agg[1] + hp_ref[...]) + b2_ref[...]


def _row_spec(d):
    return pl.BlockSpec((_BLK, d), lambda i: (i, 0))


def _dg_spec():
    return pl.BlockSpec((2, _BLK, 8), lambda i: (0, i, 0))


def _full_spec(r, cdim):
    return pl.BlockSpec((r, cdim), lambda i: (0, 0))


def _agg_spec(d):
    return pl.BlockSpec((2, _BLK, d), lambda i: (0, i, 0))


@jax.jit
def kernel(x, edge_index, W1, b1, W2, b2):
    d_hid = W1.shape[1]
    d_out = W2.shape[1]

    npad = EPAD - E
    pad = jnp.arange(npad, dtype=jnp.int32)
    src_p = jnp.concatenate([edge_index[0], pad % N]).reshape(NC * NS, K, CH)
    dst_p = jnp.concatenate([edge_index[1], N + pad % (NPAD - N)]).reshape(NC * NS, K, CH)

    ones8 = jnp.ones((CH, 8), jnp.float32)
    z8 = jnp.zeros((ZR, 8), jnp.float32)
    zh = jnp.zeros((CH, d_hid), jnp.float32)
    zo = jnp.zeros((CH, d_out), jnp.float32)

    dg = _make_deg_kernel()(dst_p, ones8, z8)

    h1p = pl.pallas_call(
        _mm_scale_body,
        grid=(_GRID,),
        in_specs=[_row_spec(x.shape[1]), _full_spec(x.shape[1], d_hid), _dg_spec()],
        out_specs=_row_spec(d_hid),
        out_shape=jax.ShapeDtypeStruct((N, d_hid), jnp.float32),
    )(x, W1, dg)

    agg1 = _make_agg_kernel(d_hid)(src_p, dst_p, h1p, zh)

    h2p = pl.pallas_call(
        _mid_body,
        grid=(_GRID,),
        in_specs=[_agg_spec(d_hid), _row_spec(d_hid), _dg_spec(),
                  _full_spec(d_hid, d_out), _full_spec(1, d_hid)],
        out_specs=_row_spec(d_out),
        out_shape=jax.ShapeDtypeStruct((N, d_out), jnp.float32),
    )(agg1, h1p, dg, W2, b1.reshape(1, d_hid))

    agg2 = _make_agg_kernel(d_out)(src_p, dst_p, h2p, zo)

    out = pl.pallas_call(
        _fin_body,
        grid=(_GRID,),
        in_specs=[_agg_spec(d_out), _row_spec(d_out), _dg_spec(),
                  _full_spec(1, d_out)],
        out_specs=_row_spec(d_out),
        out_shape=jax.ShapeDtypeStruct((N, d_out), jnp.float32),
    )(agg2, h2p, dg, b2.reshape(1, d_out))

    return out


# speedup vs baseline: 39.0132x; 1.1370x over previous
import functools

import jax
import jax.numpy as jnp
from jax import lax
from jax.experimental import pallas as pl
from jax.experimental.pallas import tpu as pltpu
from jax.experimental.pallas import tpu_sc as plsc

N = 10000
NPAD = 10240
E = 320000
NC, NS = 2, 16
CH = 128
K = 80
PH = 2
K2 = K // PH
EPAD = NC * NS * K * CH
ROWS_PER_TILE = NPAD // NS
ZR = 64


def _sc_mesh():
    return plsc.VectorSubcoreMesh(
        core_axis_name="c", subcore_axis_name="s", num_cores=NC, num_subcores=NS)


def _make_deg_kernel():
    @functools.partial(
        pl.kernel,
        out_type=jax.ShapeDtypeStruct((NC, NPAD, 8), jnp.float32),
        mesh=_sc_mesh(),
        compiler_params=pltpu.CompilerParams(use_tc_tiling_on_sc=False),
        scratch_types=[
            pltpu.VMEM((K, CH), jnp.int32),
            pltpu.VMEM((CH, 8), jnp.float32),
            pltpu.VMEM((ZR, 8), jnp.float32),
            pltpu.VMEM_SHARED((NPAD, 8), jnp.float32),
        ],
    )
    def deg_kernel(dst_hbm, ones_hbm, zz_hbm, out_hbm, dst_v, ones_v, stage_v, acc_sh):
        c = lax.axis_index("c")
        s = lax.axis_index("s")
        wid = c * NS + s
        pltpu.sync_copy(zz_hbm, stage_v)
        def zbody(i, carry):
            pltpu.sync_copy(stage_v, acc_sh.at[pl.ds(s * ROWS_PER_TILE + i * ZR, ZR)])
            return carry
        lax.fori_loop(0, ROWS_PER_TILE // ZR, zbody, 0)
        pltpu.sync_copy(ones_hbm, ones_v)
        pltpu.sync_copy(dst_hbm.at[wid], dst_v)
        plsc.subcore_barrier()
        def ebody(j, carry):
            pltpu.sync_copy(ones_v, acc_sh.at[dst_v.at[j]], add=True)
            return carry
        lax.fori_loop(0, K, ebody, 0)
        plsc.subcore_barrier()
        def obody(i, carry):
            r0 = s * ROWS_PER_TILE + i * ZR
            pltpu.sync_copy(acc_sh.at[pl.ds(r0, ZR)], stage_v)
            pltpu.sync_copy(stage_v, out_hbm.at[c, pl.ds(r0, ZR)])
            return carry
        lax.fori_loop(0, ROWS_PER_TILE // ZR, obody, 0)

    return deg_kernel


def _make_agg_kernel(d, dtype):
    @functools.partial(
        pl.kernel,
        out_type=jax.ShapeDtypeStruct((NC, NPAD, d), dtype),
        mesh=_sc_mesh(),
        compiler_params=pltpu.CompilerParams(use_tc_tiling_on_sc=False),
        scratch_types=[
            pltpu.VMEM((K2, CH), jnp.int32),
            pltpu.VMEM((K2, CH), jnp.int32),
            pltpu.VMEM((CH, d), dtype),
            pltpu.VMEM((CH, d), dtype),
            pltpu.VMEM_SHARED((NPAD, d), dtype),
            pltpu.SemaphoreType.DMA,
            pltpu.SemaphoreType.DMA,
        ],
    )
    def agg_kernel(src_hbm, dst_hbm, h_hbm, zz_hbm, out_hbm,
                   src_v, dst_v, rows_a, rows_b, acc_sh, sem_a, sem_b):
        c = lax.axis_index("c")
        s = lax.axis_index("s")
        wid = c * NS + s
        pltpu.sync_copy(zz_hbm, rows_a)
        for i in range(ROWS_PER_TILE // CH):
            pltpu.sync_copy(rows_a, acc_sh.at[pl.ds(s * ROWS_PER_TILE + i * CH, CH)])
        plsc.subcore_barrier()
        for p in range(PH):
            pltpu.sync_copy(src_hbm.at[wid, pl.ds(p * K2, K2)], src_v)
            pltpu.sync_copy(dst_hbm.at[wid, pl.ds(p * K2, K2)], dst_v)
            pltpu.async_copy(h_hbm.at[src_v.at[0]], rows_a, sem_a)
            def ebody(i, carry):
                c0 = 2 * i
                c1 = 2 * i + 1
                pltpu.async_copy(h_hbm.at[src_v.at[c1]], rows_b, sem_b)
                pltpu.make_async_copy(h_hbm.at[src_v.at[c0]], rows_a, sem_a).wait()
                pltpu.sync_copy(rows_a, acc_sh.at[dst_v.at[c0]], add=True)
                nxt = jnp.minimum(c0 + 2, K2 - 1)
                pltpu.async_copy(h_hbm.at[src_v.at[nxt]], rows_a, sem_a)
                pltpu.make_async_copy(h_hbm.at[src_v.at[c1]], rows_b, sem_b).wait()
                pltpu.sync_copy(rows_b, acc_sh.at[dst_v.at[c1]], add=True)
                return carry
            lax.fori_loop(0, K2 // 2, ebody, 0)
            pltpu.make_async_copy(h_hbm.at[src_v.at[K2 - 1]], rows_a, sem_a).wait()
        plsc.subcore_barrier()
        for i in range(ROWS_PER_TILE // CH):
            r0 = s * ROWS_PER_TILE + i * CH
            pltpu.sync_copy(acc_sh.at[pl.ds(r0, CH)], rows_a)
            pltpu.sync_copy(rows_a, out_hbm.at[c, pl.ds(r0, CH)])

    return agg_kernel


_BLK = 2000
_GRID = N // _BLK


def _dis_from(dg):
    deg = dg[0, :, 0] + dg[1, :, 0] + 1.0
    return lax.rsqrt(deg)


def _mm_scale_body(x_ref, w_ref, dg_ref, o_ref):
    dis = _dis_from(dg_ref[...])
    h = jnp.dot(x_ref[...], w_ref[...], preferred_element_type=jnp.float32)
    o_ref[...] = (h * dis[:, None]).astype(o_ref.dtype)


def _mid_body(agg_ref, hp_ref, dg_ref, w2_ref, b1_ref, o_ref):
    dis = _dis_from(dg_ref[...])
    agg = agg_ref[...].astype(jnp.float32)
    hp = hp_ref[...].astype(jnp.float32)
    h1 = dis[:, None] * (agg[0] + agg[1] + hp) + b1_ref[...]
    h1 = jnp.maximum(h1, 0.0)
    h2 = jnp.dot(h1, w2_ref[...], preferred_element_type=jnp.float32) * dis[:, None]
    o_ref[...] = h2.astype(o_ref.dtype)


def _fin_body(agg_ref, hp_ref, dg_ref, b2_ref, o_ref):
    dis = _dis_from(dg_ref[...])
    agg = agg_ref[...].astype(jnp.float32)
    hp = hp_ref[...].astype(jnp.float32)
    o_ref[...] = dis[:, None] * (agg[0] + agg[1] + hp) + b2_ref[...]


def _row_spec(d):
    return pl.BlockSpec((_BLK, d), lambda i: (i, 0))


def _dg_spec():
    return pl.BlockSpec((2, _BLK, 8), lambda i: (0, i, 0))


def _full_spec(r, cdim):
    return pl.BlockSpec((r, cdim), lambda i: (0, 0))


def _agg_spec(d):
    return pl.BlockSpec((2, _BLK, d), lambda i: (0, i, 0))


@jax.jit
def kernel(x, edge_index, W1, b1, W2, b2):
    d_hid = W1.shape[1]
    d_out = W2.shape[1]

    npad = EPAD - E
    pad = jnp.arange(npad, dtype=jnp.int32)
    src_p = jnp.concatenate([edge_index[0], pad % N]).reshape(NC * NS, K, CH)
    dst_p = jnp.concatenate([edge_index[1], N + pad % (NPAD - N)]).reshape(NC * NS, K, CH)

    ones8 = jnp.ones((CH, 8), jnp.float32)
    z8 = jnp.zeros((ZR, 8), jnp.float32)
    zh = jnp.zeros((CH, d_hid), jnp.bfloat16)
    zo = jnp.zeros((CH, d_out), jnp.bfloat16)

    dg = _make_deg_kernel()(dst_p, ones8, z8)

    h1p = pl.pallas_call(
        _mm_scale_body,
        grid=(_GRID,),
        in_specs=[_row_spec(x.shape[1]), _full_spec(x.shape[1], d_hid), _dg_spec()],
        out_specs=_row_spec(d_hid),
        out_shape=jax.ShapeDtypeStruct((N, d_hid), jnp.bfloat16),
    )(x, W1, dg)

    agg1 = _make_agg_kernel(d_hid, jnp.bfloat16)(src_p, dst_p, h1p, zh)

    h2p = pl.pallas_call(
        _mid_body,
        grid=(_GRID,),
        in_specs=[_agg_spec(d_hid), _row_spec(d_hid), _dg_spec(),
                  _full_spec(d_hid, d_out), _full_spec(1, d_hid)],
        out_specs=_row_spec(d_out),
        out_shape=jax.ShapeDtypeStruct((N, d_out), jnp.bfloat16),
    )(agg1, h1p, dg, W2, b1.reshape(1, d_hid))

    agg2 = _make_agg_kernel(d_out, jnp.bfloat16)(src_p, dst_p, h2p, zo)

    out = pl.pallas_call(
        _fin_body,
        grid=(_GRID,),
        in_specs=[_agg_spec(d_out), _row_spec(d_out), _dg_spec(),
                  _full_spec(1, d_out)],
        out_specs=_row_spec(d_out),
        out_shape=jax.ShapeDtypeStruct((N, d_out), jnp.float32),
    )(agg2, h2p, dg, b2.reshape(1, d_out))

    return out
